# Initial kernel scaffold; baseline (speedup 1.0000x reference)
import jax
import jax.numpy as jnp
from jax.experimental import pallas as pl


def kernel(z, embedding):
    raise NotImplementedError("write your pallas kernel here")


# speedup vs baseline: 1.9730x; 1.9730x over previous
import jax
import jax.numpy as jnp
from jax.experimental import pallas as pl
from jax.experimental.pallas import tpu as pltpu

B, D, HW = 8, 256, 1024
K = 1024
BETA = 0.25
_LOSS_SCALE = (1.0 + BETA) / (B * HW * D)


def _vq_body(z_ref, e_ref, zq_ref, idx_ref, loss_ref):
    b = pl.program_id(0)

    e = e_ref[...]
    es = jnp.sum(e * e, axis=1, keepdims=True)
    en = e * (1.0 / jnp.maximum(jnp.sqrt(es), 1e-12))
    e_sq = jnp.sum(en * en, axis=1, keepdims=True)

    zb = z_ref[0]
    s = jnp.sum(zb * zb, axis=0, keepdims=True)
    zn = zb * (1.0 / jnp.maximum(jnp.sqrt(s), 1e-12))
    znsq = jnp.sum(zn * zn, axis=0, keepdims=True)

    scores = jnp.dot(en, zn, preferred_element_type=jnp.float32)
    dist = e_sq + znsq - 2.0 * scores

    minv = jnp.min(dist, axis=0, keepdims=True)
    iota_c = jax.lax.broadcasted_iota(jnp.int32, (K, HW), 0)
    idx = jnp.min(jnp.where(dist == minv, iota_c, 2 ** 30), axis=0,
                  keepdims=True)
    idx_ref[0] = idx

    onehot = (iota_c == idx).astype(jnp.float32)
    zq = jax.lax.dot_general(en, onehot, (((0,), (0,)), ((), ())),
                             preferred_element_type=jnp.float32)
    zq_ref[0] = zq

    @pl.when(b == 0)
    def _():
        loss_ref[0, 0] = 0.0

    loss_ref[0, 0] += jnp.sum(minv) * _LOSS_SCALE


def kernel(z, embedding):
    z3 = z.reshape(B, D, HW)
    zq, idx, loss = pl.pallas_call(
        _vq_body,
        grid=(B,),
        in_specs=[
            pl.BlockSpec((1, D, HW), lambda b: (b, 0, 0)),
            pl.BlockSpec((K, D), lambda b: (0, 0)),
        ],
        out_specs=[
            pl.BlockSpec((1, D, HW), lambda b: (b, 0, 0)),
            pl.BlockSpec((1, 1, HW), lambda b: (b, 0, 0)),
            pl.BlockSpec((1, 1), lambda b: (0, 0), memory_space=pltpu.SMEM),
        ],
        out_shape=[
            jax.ShapeDtypeStruct((B, D, HW), jnp.float32),
            jax.ShapeDtypeStruct((B, 1, HW), jnp.int32),
            jax.ShapeDtypeStruct((1, 1), jnp.float32),
        ],
    )(z3, embedding)
    return (zq.reshape(B, D, 32, 32), idx.reshape(B * HW), loss[0, 0])
